# baseline (speedup 1.0000x reference)
import jax
import jax.numpy as jnp
from jax.experimental import pallas as pl
from jax.experimental.pallas import tpu as pltpu

S = 2048
D = 768
H = 12
HD = 64
K = 64
BLK = 128
WIN = 192
NBLK = S // BLK


def _fused_kernel(x_ref, win_ref, wout_ref, bout_ref, out_ref, h_ref, f_ref):
    h_ref[...] = jnp.dot(x_ref[...].astype(jnp.bfloat16), win_ref[...],
                         preferred_element_type=jnp.float32
                         ).astype(jnp.bfloat16)

    inv_sqrt_hd = 1.0 / jnp.sqrt(jnp.float32(HD))

    def body(i, carry):
        r0 = i * BLK
        c0 = pl.multiple_of(jnp.clip(r0 - K // 2, 0, S - WIN), 32)
        q = h_ref[pl.ds(r0, BLK), :]
        kw = h_ref[pl.ds(c0, WIN), :]

        rows = r0 + jax.lax.broadcasted_iota(jnp.int32, (BLK, WIN), 0)
        cols = c0 + jax.lax.broadcasted_iota(jnp.int32, (BLK, WIN), 1)
        off = cols - rows
        valid = (off >= -(K // 2)) & (off <= K // 2 - 1)
        rowsf = rows.astype(jnp.float32)
        cnt = jnp.where(cols == 0, jnp.maximum(33.0 - rowsf, 1.0), 1.0)
        cnt = jnp.where(cols == S - 1,
                        jnp.maximum(rowsf - (S - 33.0), 1.0), cnt)
        bias = jnp.where(valid, jnp.log(cnt), -1e30)

        for hd in range(H):
            qh = q[:, hd * HD:(hd + 1) * HD]
            kh = kw[:, hd * HD:(hd + 1) * HD]
            s = jax.lax.dot_general(
                qh, kh, (((1,), (1,)), ((), ())),
                preferred_element_type=jnp.float32) * inv_sqrt_hd + bias
            m = jnp.max(s, axis=1, keepdims=True)
            p = jnp.exp(s - m)
            z = jnp.sum(p, axis=1, keepdims=True)
            f_ref[:, hd * HD:(hd + 1) * HD] = (jnp.dot(
                p.astype(jnp.bfloat16), kh,
                preferred_element_type=jnp.float32) / z
            ).astype(jnp.bfloat16)

        out_ref[pl.ds(r0, BLK), :] = (
            jnp.dot(f_ref[...], wout_ref[...],
                    preferred_element_type=jnp.float32)
            + bout_ref[...] + x_ref[pl.ds(r0, BLK), :])
        return carry

    jax.lax.fori_loop(0, NBLK, body, 0)


@jax.jit
def kernel(x, W_in, W_out, b_out):
    x2 = x.reshape(S, D)
    out = pl.pallas_call(
        _fused_kernel,
        out_shape=jax.ShapeDtypeStruct((S, D), jnp.float32),
        scratch_shapes=[
            pltpu.VMEM((S, D), jnp.bfloat16),
            pltpu.VMEM((BLK, D), jnp.bfloat16),
        ],
    )(x2, W_in.astype(jnp.bfloat16), W_out.astype(jnp.bfloat16),
      b_out.reshape(1, D))
    return out.reshape(1, S, D)


# speedup vs baseline: 1.0517x; 1.0517x over previous
import jax
import jax.numpy as jnp
from jax.experimental import pallas as pl
from jax.experimental.pallas import tpu as pltpu

S = 2048
D = 768
H = 12
HD = 64
K = 64
BLK = 128
WIN = 192
NBLK = S // BLK


def _fused_kernel(x_ref, win_ref, wout_ref, bout_ref, out_ref, h_ref, f_ref):
    h_ref[...] = jnp.dot(x_ref[...].astype(jnp.bfloat16), win_ref[...],
                         preferred_element_type=jnp.float32
                         ).astype(jnp.bfloat16)

    inv_sqrt_hd = 1.0 / jnp.sqrt(jnp.float32(HD))

    def body(i, carry):
        r0 = i * BLK
        c0 = pl.multiple_of(jnp.clip(r0 - K // 2, 0, S - WIN), 32)
        q = h_ref[pl.ds(r0, BLK), :]
        kw = h_ref[pl.ds(c0, WIN), :]

        rows = r0 + jax.lax.broadcasted_iota(jnp.int32, (BLK, WIN), 0)
        cols = c0 + jax.lax.broadcasted_iota(jnp.int32, (BLK, WIN), 1)
        off = cols - rows
        valid = (off >= -(K // 2)) & (off <= K // 2 - 1)
        rowsf = rows.astype(jnp.float32)
        cnt = jnp.where(cols == 0, jnp.maximum(33.0 - rowsf, 1.0), 1.0)
        cnt = jnp.where(cols == S - 1,
                        jnp.maximum(rowsf - (S - 33.0), 1.0), cnt)
        bias = jnp.where(valid, jnp.log(cnt), -1e30)

        for hd in range(H):
            qh = q[:, hd * HD:(hd + 1) * HD]
            kh = kw[:, hd * HD:(hd + 1) * HD]
            s = jax.lax.dot_general(
                qh, kh, (((1,), (1,)), ((), ())),
                preferred_element_type=jnp.float32) * inv_sqrt_hd + bias
            m = jnp.max(s, axis=1, keepdims=True)
            p = jnp.exp(s - m)
            z = jnp.sum(p, axis=1, keepdims=True)
            f_ref[pl.ds(r0, BLK), hd * HD:(hd + 1) * HD] = (jnp.dot(
                p.astype(jnp.bfloat16), kh,
                preferred_element_type=jnp.float32) / z
            ).astype(jnp.bfloat16)

        return carry

    jax.lax.fori_loop(0, NBLK, body, 0)

    out_ref[...] = (jnp.dot(f_ref[...], wout_ref[...],
                            preferred_element_type=jnp.float32)
                    + bout_ref[...] + x_ref[...])


@jax.jit
def kernel(x, W_in, W_out, b_out):
    x2 = x.reshape(S, D)
    out = pl.pallas_call(
        _fused_kernel,
        out_shape=jax.ShapeDtypeStruct((S, D), jnp.float32),
        scratch_shapes=[
            pltpu.VMEM((S, D), jnp.bfloat16),
            pltpu.VMEM((S, D), jnp.bfloat16),
        ],
    )(x2, W_in.astype(jnp.bfloat16), W_out.astype(jnp.bfloat16),
      b_out.reshape(1, D))
    return out.reshape(1, S, D)
